# Initial kernel scaffold; baseline (speedup 1.0000x reference)
import jax
import jax.numpy as jnp
from jax.experimental import pallas as pl


def kernel(x, edge_index, W1, b1, W2, b2):
    raise NotImplementedError("write your pallas kernel here")


# speedup vs baseline: 1.9259x; 1.9259x over previous
import jax
import jax.numpy as jnp
from jax import lax
from jax.experimental import pallas as pl
from jax.experimental.pallas import tpu as pltpu
from jax.experimental.pallas import tpu_sc as plsc

N = 10000
E = 320000
D_HID = 128
D_OUT = 64

NC, NS = 2, 16
NW = NC * NS
EPW = 10112
E_PAD = NW * EPW
HALF = E_PAD // NC
B = 2048
NCHP = HALF // B
NPAD = 10112
TRASH = N

_mesh = plsc.VectorSubcoreMesh(core_axis_name="c", subcore_axis_name="s")
_sc_params = pltpu.CompilerParams(needs_layout_passes=False,
                                  use_tc_tiling_on_sc=False)


def _prep_body(src_hbm, dst_hbm, dstp_hbm, degp_hbm, srcb, dstb, dstpb, dacc):
    c = lax.axis_index("c")
    s = lax.axis_index("s")
    wid = s * NC + c
    e0 = wid * EPW
    pltpu.sync_copy(src_hbm.at[pl.ds(e0, EPW)], srcb)
    pltpu.sync_copy(dst_hbm.at[pl.ds(e0, EPW)], dstb)

    z16 = jnp.zeros((16,), jnp.float32)

    @pl.loop(0, NPAD // 16)
    def _(i):
        dacc[pl.ds(i * 16, 16)] = z16

    ones = jnp.full((16,), 1.0, jnp.float32)

    @pl.loop(0, EPW // 16)
    def _(i):
        sl = pl.ds(i * 16, 16)
        sv = srcb[sl]
        dv = dstb[sl]
        m = sv == dv
        dstpb[sl] = jnp.where(m, TRASH, dv)
        srcp = jnp.where(m, TRASH, sv)
        plsc.addupdate_scatter(dacc, [srcp], ones)

    pltpu.sync_copy(dstpb, dstp_hbm.at[pl.ds(e0, EPW)])
    pltpu.sync_copy(dacc, degp_hbm.at[wid])


_prep = pl.kernel(
    _prep_body,
    out_type=[
        jax.ShapeDtypeStruct((E_PAD,), jnp.int32),
        jax.ShapeDtypeStruct((NW, NPAD), jnp.float32),
    ],
    mesh=_mesh,
    compiler_params=_sc_params,
    scratch_types=[
        pltpu.VMEM((EPW,), jnp.int32),
        pltpu.VMEM((EPW,), jnp.int32),
        pltpu.VMEM((EPW,), jnp.int32),
        pltpu.VMEM((NPAD,), jnp.float32),
    ],
)


def _prop_body(u3_hbm, src_hbm, dstp_hbm, z8_hbm, aggp_hbm,
               idx_s, idx_d, idx_a, strip, acc, gsem):
    c = lax.axis_index("c")
    s = lax.axis_index("s")
    pltpu.sync_copy(z8_hbm, acc)
    toff = s * N
    iota = lax.broadcasted_iota(jnp.int32, (16,), 0)
    kvecs = [jnp.full((16,), k, jnp.int32) for k in range(8)]

    @pl.loop(0, NCHP)
    def _(j):
        off = c * HALF + j * B
        pltpu.sync_copy(src_hbm.at[pl.ds(off, B)], idx_s)
        pltpu.sync_copy(dstp_hbm.at[pl.ds(off, B)], idx_d)

        @pl.loop(0, B // 16)
        def _(g):
            sl = pl.ds(g * 16, 16)
            idx_a[sl] = idx_s[sl] + toff

        descs = [
            pltpu.async_copy(
                u3_hbm.at[idx_a.at[pl.ds(i * 128, 128)]],
                strip.at[pl.ds(i * 128, 128)],
                gsem,
            )
            for i in range(B // 128)
        ]
        for dsc in descs:
            dsc.wait()

        @pl.loop(0, B // 16)
        def _(g):
            rv = iota + g * 16
            dstv = idx_d[pl.ds(g * 16, 16)]
            for k in range(8):
                v = plsc.load_gather(strip, [rv, kvecs[k]])
                plsc.addupdate_scatter(acc, [dstv, kvecs[k]], v)

    pltpu.sync_copy(acc, aggp_hbm.at[c, s])


_prop = pl.kernel(
    _prop_body,
    out_type=[jax.ShapeDtypeStruct((NC, NS, NPAD, 8), jnp.float32)],
    mesh=_mesh,
    compiler_params=_sc_params,
    scratch_types=[
        pltpu.VMEM((B,), jnp.int32),
        pltpu.VMEM((B,), jnp.int32),
        pltpu.VMEM((B,), jnp.int32),
        pltpu.VMEM((B, 8), jnp.float32),
        pltpu.VMEM((NPAD, 8), jnp.float32),
        pltpu.SemaphoreType.DMA,
    ],
)


R = 2000
NBLK = N // R


def _dis_from(degp_ref):
    deg = jnp.sum(degp_ref[...], axis=(0, 1, 2))[:, None]
    return jnp.where(deg > 0, 1.0 / jnp.sqrt(jnp.where(deg > 0, deg, 1.0)), 0.0)


def _pre1_body(degp_ref, x_ref, w_ref, b_ref, u0_ref, y0_ref):
    dis = _dis_from(degp_ref)
    xb = x_ref[...]
    u0_ref[...] = dis * xb
    w02 = w_ref[0] - w_ref[2]
    y0_ref[...] = jnp.dot(xb, w02, preferred_element_type=jnp.float32) + b_ref[...]


def _mid_body(degp_ref, ap_ref, y_ref, w_ref, u1_ref, y1_ref):
    dis = _dis_from(degp_ref)
    a = ap_ref[0] + ap_ref[1]
    tx1 = -dis * a
    u1_ref[...] = dis * tx1
    y1_ref[...] = y_ref[...] + jnp.dot(tx1, w_ref[1],
                                       preferred_element_type=jnp.float32)


def _fin1_body(degp_ref, ap_ref, y_ref, w1_ref, w2_ref, b2_ref,
               u0_ref, y0_ref):
    dis = _dis_from(degp_ref)
    p2 = -dis * (ap_ref[0] + ap_ref[1])
    h = jnp.maximum(
        y_ref[...] + 2.0 * jnp.dot(p2, w1_ref[2],
                                   preferred_element_type=jnp.float32), 0.0)
    u0_ref[...] = dis * h
    w02 = w2_ref[0] - w2_ref[2]
    y0_ref[...] = jnp.dot(h, w02, preferred_element_type=jnp.float32) + b2_ref[...]


def _fin2_body(degp_ref, ap_ref, y_ref, w2_ref, out_ref):
    dis = _dis_from(degp_ref)
    p2 = -dis * (ap_ref[0] + ap_ref[1])
    o = y_ref[...] + 2.0 * jnp.dot(p2, w2_ref[2],
                                   preferred_element_type=jnp.float32)
    m = jnp.max(o, axis=1, keepdims=True)
    e = jnp.exp(o - m)
    out_ref[...] = e / jnp.sum(e, axis=1, keepdims=True)


def _bs_deg():
    return pl.BlockSpec((NW, 1, 1, R), lambda i: (0, i, 0, 0))


def _bs_agg():
    return pl.BlockSpec((NC, R, 128), lambda i: (0, i, 0))


def _bs_rows(d):
    return pl.BlockSpec((R, d), lambda i: (i, 0))


def _bs_full(shape):
    nd = len(shape)
    return pl.BlockSpec(shape, lambda i, _nd=nd: (0,) * _nd)


def _rows_out(d):
    return jax.ShapeDtypeStruct((N, d), jnp.float32)


def _to_strips(u):
    return jnp.transpose(u.reshape(N, NS, 8), (1, 0, 2)).reshape(NS * N, 8)


def _from_strips(aggp):
    return jnp.transpose(aggp, (0, 2, 1, 3)).reshape(NC, NPAD, 128)


def kernel(x, edge_index, W1, b1, W2, b2):
    pad = jnp.zeros((E_PAD - E,), jnp.int32)
    src_p = jnp.concatenate([edge_index[0], pad])
    dst_p = jnp.concatenate([edge_index[1], pad])
    z8 = jnp.zeros((NPAD, 8), jnp.float32)
    b1r = b1.reshape(1, D_HID)
    b2r = b2.reshape(1, D_OUT)

    dstp, degp = _prep(src_p, dst_p)
    degp = degp[:, :N].reshape(NW, NBLK, 1, R)

    u0, y0 = pl.pallas_call(
        _pre1_body,
        grid=(NBLK,),
        in_specs=[_bs_deg(), _bs_rows(128), _bs_full((3, 128, 128)),
                  _bs_full((1, 128))],
        out_specs=[_bs_rows(128), _bs_rows(128)],
        out_shape=[_rows_out(128), _rows_out(128)],
    )(degp, x, W1, b1r)

    def prop(u):
        (aggp,) = _prop(_to_strips(u), src_p, dstp, z8)
        return _from_strips(aggp)

    def mid(ap, y, W, do):
        return pl.pallas_call(
            _mid_body,
            grid=(NBLK,),
            in_specs=[_bs_deg(), _bs_agg(), _bs_rows(do), _bs_full(W.shape)],
            out_specs=[_bs_rows(128), _bs_rows(do)],
            out_shape=[_rows_out(128), _rows_out(do)],
        )(degp, ap, y, W)

    a1 = prop(u0)
    u1, y1 = mid(a1, y0, W1, 128)
    a2 = prop(u1)

    u0b, y0b = pl.pallas_call(
        _fin1_body,
        grid=(NBLK,),
        in_specs=[_bs_deg(), _bs_agg(), _bs_rows(128),
                  _bs_full((3, 128, 128)), _bs_full((3, 128, 64)),
                  _bs_full((1, 64))],
        out_specs=[_bs_rows(128), _bs_rows(64)],
        out_shape=[_rows_out(128), _rows_out(64)],
    )(degp, a2, y1, W1, W2, b2r)

    a1b = prop(u0b)
    u1b, y1b = mid(a1b, y0b, W2, 64)
    a2b = prop(u1b)

    out = pl.pallas_call(
        _fin2_body,
        grid=(NBLK,),
        in_specs=[_bs_deg(), _bs_agg(), _bs_rows(64), _bs_full((3, 128, 64))],
        out_specs=_bs_rows(64),
        out_shape=_rows_out(64),
    )(degp, a2b, y1b, W2)
    return out


# speedup vs baseline: 2.3257x; 1.2076x over previous
import jax
import jax.numpy as jnp
from jax import lax
from jax.experimental import pallas as pl
from jax.experimental.pallas import tpu as pltpu
from jax.experimental.pallas import tpu_sc as plsc

N = 10000
E = 320000
D_HID = 128
D_OUT = 64

NC, NS = 2, 16
NW = NC * NS
EPW = 10112
E_PAD = NW * EPW
HALF = E_PAD // NC
B = 2048
NCHP = HALF // B
NPAD = 10112
TRASH = N

_mesh = plsc.VectorSubcoreMesh(core_axis_name="c", subcore_axis_name="s")
_sc_params = pltpu.CompilerParams(needs_layout_passes=False,
                                  use_tc_tiling_on_sc=False)


def _prep_body(src_hbm, dst_hbm, dstp_hbm, degp_hbm, srcb, dstb, dstpb, dacc):
    c = lax.axis_index("c")
    s = lax.axis_index("s")
    wid = s * NC + c
    e0 = wid * EPW
    pltpu.sync_copy(src_hbm.at[pl.ds(e0, EPW)], srcb)
    pltpu.sync_copy(dst_hbm.at[pl.ds(e0, EPW)], dstb)

    z16 = jnp.zeros((16,), jnp.float32)

    @pl.loop(0, NPAD // 16)
    def _(i):
        dacc[pl.ds(i * 16, 16)] = z16

    ones = jnp.full((16,), 1.0, jnp.float32)

    @pl.loop(0, EPW // 16)
    def _(i):
        sl = pl.ds(i * 16, 16)
        sv = srcb[sl]
        dv = dstb[sl]
        m = sv == dv
        dstpb[sl] = jnp.where(m, TRASH, dv)
        srcp = jnp.where(m, TRASH, sv)
        plsc.addupdate_scatter(dacc, [srcp], ones)

    pltpu.sync_copy(dstpb, dstp_hbm.at[pl.ds(e0, EPW)])
    pltpu.sync_copy(dacc, degp_hbm.at[wid])


_prep = pl.kernel(
    _prep_body,
    out_type=[
        jax.ShapeDtypeStruct((E_PAD,), jnp.int32),
        jax.ShapeDtypeStruct((NW, NPAD), jnp.float32),
    ],
    mesh=_mesh,
    compiler_params=_sc_params,
    scratch_types=[
        pltpu.VMEM((EPW,), jnp.int32),
        pltpu.VMEM((EPW,), jnp.int32),
        pltpu.VMEM((EPW,), jnp.int32),
        pltpu.VMEM((NPAD,), jnp.float32),
    ],
)


def _prop_body(u3_hbm, src_hbm, dstp_hbm, z8_hbm, aggp_hbm,
               is0, id0, st0, is1, id1, st1, acc, sem0, sem1):
    c = lax.axis_index("c")
    s = lax.axis_index("s")
    pltpu.sync_copy(z8_hbm, acc)
    toff = s * N
    iota = lax.broadcasted_iota(jnp.int32, (16,), 0)
    kvecs = [jnp.full((16,), k, jnp.int32) for k in range(8)]

    def descs(isr, stripb, sem):
        return [
            pltpu.make_async_copy(
                u3_hbm.at[isr.at[pl.ds(i * 128, 128)]],
                stripb.at[pl.ds(i * 128, 128)],
                sem,
            )
            for i in range(B // 128)
        ]

    def fire(j, isr, idr, stripb, sem):
        off = c * HALF + j * B
        pltpu.sync_copy(src_hbm.at[pl.ds(off, B)], isr)
        pltpu.sync_copy(dstp_hbm.at[pl.ds(off, B)], idr)

        @pl.loop(0, B // 16)
        def _(g):
            sl = pl.ds(g * 16, 16)
            isr[sl] = isr[sl] + toff

        for d in descs(isr, stripb, sem):
            d.start()

    def drain(isr, stripb, sem):
        for d in descs(isr, stripb, sem):
            d.wait()

    def accum(idr, stripb):
        @pl.loop(0, B // 16)
        def _(g):
            rv = iota + g * 16
            dstv = idr[pl.ds(g * 16, 16)]
            for k in range(8):
                v = plsc.load_gather(stripb, [rv, kvecs[k]])
                plsc.addupdate_scatter(acc, [dstv, kvecs[k]], v)

    fire(0, is0, id0, st0, sem0)

    @pl.loop(0, (NCHP - 1) // 2)
    def _(p):
        j = 2 * p
        drain(is0, st0, sem0)
        fire(j + 1, is1, id1, st1, sem1)
        accum(id0, st0)
        drain(is1, st1, sem1)
        fire(j + 2, is0, id0, st0, sem0)
        accum(id1, st1)

    drain(is0, st0, sem0)
    accum(id0, st0)

    pltpu.sync_copy(acc, aggp_hbm.at[c, s])


_prop = pl.kernel(
    _prop_body,
    out_type=[jax.ShapeDtypeStruct((NC, NS, NPAD, 8), jnp.float32)],
    mesh=_mesh,
    compiler_params=_sc_params,
    scratch_types=[
        pltpu.VMEM((B,), jnp.int32),
        pltpu.VMEM((B,), jnp.int32),
        pltpu.VMEM((B, 8), jnp.float32),
        pltpu.VMEM((B,), jnp.int32),
        pltpu.VMEM((B,), jnp.int32),
        pltpu.VMEM((B, 8), jnp.float32),
        pltpu.VMEM((NPAD, 8), jnp.float32),
        pltpu.SemaphoreType.DMA,
        pltpu.SemaphoreType.DMA,
    ],
)


R = 2000
NBLK = N // R


def _dis_from(degp_ref):
    deg = jnp.sum(degp_ref[...], axis=(0, 1, 2))[:, None]
    return jnp.where(deg > 0, 1.0 / jnp.sqrt(jnp.where(deg > 0, deg, 1.0)), 0.0)


def _pre1_body(degp_ref, x_ref, w_ref, b_ref, u0_ref, y0_ref):
    dis = _dis_from(degp_ref)
    xb = x_ref[...]
    u0_ref[...] = dis * xb
    w02 = w_ref[0] - w_ref[2]
    y0_ref[...] = jnp.dot(xb, w02, preferred_element_type=jnp.float32) + b_ref[...]


def _mid_body(degp_ref, ap_ref, y_ref, w_ref, u1_ref, y1_ref):
    dis = _dis_from(degp_ref)
    a = ap_ref[0] + ap_ref[1]
    tx1 = -dis * a
    u1_ref[...] = dis * tx1
    y1_ref[...] = y_ref[...] + jnp.dot(tx1, w_ref[1],
                                       preferred_element_type=jnp.float32)


def _fin1_body(degp_ref, ap_ref, y_ref, w1_ref, w2_ref, b2_ref,
               u0_ref, y0_ref):
    dis = _dis_from(degp_ref)
    p2 = -dis * (ap_ref[0] + ap_ref[1])
    h = jnp.maximum(
        y_ref[...] + 2.0 * jnp.dot(p2, w1_ref[2],
                                   preferred_element_type=jnp.float32), 0.0)
    u0_ref[...] = dis * h
    w02 = w2_ref[0] - w2_ref[2]
    y0_ref[...] = jnp.dot(h, w02, preferred_element_type=jnp.float32) + b2_ref[...]


def _fin2_body(degp_ref, ap_ref, y_ref, w2_ref, out_ref):
    dis = _dis_from(degp_ref)
    p2 = -dis * (ap_ref[0] + ap_ref[1])
    o = y_ref[...] + 2.0 * jnp.dot(p2, w2_ref[2],
                                   preferred_element_type=jnp.float32)
    m = jnp.max(o, axis=1, keepdims=True)
    e = jnp.exp(o - m)
    out_ref[...] = e / jnp.sum(e, axis=1, keepdims=True)


def _bs_deg():
    return pl.BlockSpec((NW, 1, 1, R), lambda i: (0, i, 0, 0))


def _bs_agg():
    return pl.BlockSpec((NC, R, 128), lambda i: (0, i, 0))


def _bs_rows(d):
    return pl.BlockSpec((R, d), lambda i: (i, 0))


def _bs_full(shape):
    nd = len(shape)
    return pl.BlockSpec(shape, lambda i, _nd=nd: (0,) * _nd)


def _rows_out(d):
    return jax.ShapeDtypeStruct((N, d), jnp.float32)


def _to_strips(u):
    return jnp.transpose(u.reshape(N, NS, 8), (1, 0, 2)).reshape(NS * N, 8)


def _from_strips(aggp):
    return jnp.transpose(aggp, (0, 2, 1, 3)).reshape(NC, NPAD, 128)


def kernel(x, edge_index, W1, b1, W2, b2):
    pad = jnp.zeros((E_PAD - E,), jnp.int32)
    src_p = jnp.concatenate([edge_index[0], pad])
    dst_p = jnp.concatenate([edge_index[1], pad])
    z8 = jnp.zeros((NPAD, 8), jnp.float32)
    b1r = b1.reshape(1, D_HID)
    b2r = b2.reshape(1, D_OUT)

    dstp, degp = _prep(src_p, dst_p)
    degp = degp[:, :N].reshape(NW, NBLK, 1, R)

    u0, y0 = pl.pallas_call(
        _pre1_body,
        grid=(NBLK,),
        in_specs=[_bs_deg(), _bs_rows(128), _bs_full((3, 128, 128)),
                  _bs_full((1, 128))],
        out_specs=[_bs_rows(128), _bs_rows(128)],
        out_shape=[_rows_out(128), _rows_out(128)],
    )(degp, x, W1, b1r)

    def prop(u):
        (aggp,) = _prop(_to_strips(u), src_p, dstp, z8)
        return _from_strips(aggp)

    def mid(ap, y, W, do):
        return pl.pallas_call(
            _mid_body,
            grid=(NBLK,),
            in_specs=[_bs_deg(), _bs_agg(), _bs_rows(do), _bs_full(W.shape)],
            out_specs=[_bs_rows(128), _bs_rows(do)],
            out_shape=[_rows_out(128), _rows_out(do)],
        )(degp, ap, y, W)

    a1 = prop(u0)
    u1, y1 = mid(a1, y0, W1, 128)
    a2 = prop(u1)

    u0b, y0b = pl.pallas_call(
        _fin1_body,
        grid=(NBLK,),
        in_specs=[_bs_deg(), _bs_agg(), _bs_rows(128),
                  _bs_full((3, 128, 128)), _bs_full((3, 128, 64)),
                  _bs_full((1, 64))],
        out_specs=[_bs_rows(128), _bs_rows(64)],
        out_shape=[_rows_out(128), _rows_out(64)],
    )(degp, a2, y1, W1, W2, b2r)

    a1b = prop(u0b)
    u1b, y1b = mid(a1b, y0b, W2, 64)
    a2b = prop(u1b)

    out = pl.pallas_call(
        _fin2_body,
        grid=(NBLK,),
        in_specs=[_bs_deg(), _bs_agg(), _bs_rows(64), _bs_full((3, 128, 64))],
        out_specs=_bs_rows(64),
        out_shape=_rows_out(64),
    )(degp, a2b, y1b, W2)
    return out
